# baseline (speedup 1.0000x reference)
import jax
import jax.numpy as jnp
from jax.experimental import pallas as pl
from jax.experimental.pallas import tpu as pltpu


def _rvq_body(emb_ref, cb_ref, stack_ref, hcsq_ref, out_ref, resid_ref):
    nq = pl.num_programs(0)
    q = pl.program_id(0)
    i = pl.program_id(1)
    tn = out_ref.shape[2]
    k = cb_ref.shape[1]
    rows = pl.ds(i * tn, tn)

    @pl.when(q == 0)
    def _():
        resid_ref[rows, :] = emb_ref[rows, :]

    def mm(a, b, contract_b):
        return jax.lax.dot_general(
            a, b, (((1,), (contract_b,)), ((), ())),
            preferred_element_type=jnp.float32,
        )

    resid = resid_ref[rows, :]
    dots = mm(resid, cb_ref[0], 1)
    score = dots - hcsq_ref[q][None, :]
    idx = jnp.argmax(score, axis=1).astype(jnp.int32)
    out_ref[0, 0, :] = idx

    @pl.when(q < nq - 1)
    def _():
        iota3 = jax.lax.broadcasted_iota(jnp.int32, (tn, 3 * k), 1)
        onehot3 = ((iota3 & (k - 1)) == idx[:, None]).astype(jnp.bfloat16)
        quant = mm(onehot3, stack_ref[0], 0)
        resid_ref[rows, :] = resid - quant


def kernel(embedding, codebooks):
    n, d = embedding.shape
    nq, k, _ = codebooks.shape
    tn = min(1024, n)
    grid_n = n // tn

    half_csq = 0.5 * jnp.sum(codebooks * codebooks, axis=-1)

    def trunc_hi16(x):
        return jax.lax.bitcast_convert_type(
            jax.lax.bitcast_convert_type(x, jnp.uint32) & jnp.uint32(0xFFFF0000),
            jnp.float32)

    hi_v = trunc_hi16(codebooks)
    r1 = codebooks - hi_v
    mid_v = trunc_hi16(r1)
    lo_v = r1 - mid_v
    cb_stack = jnp.concatenate(
        [hi_v.astype(jnp.bfloat16), mid_v.astype(jnp.bfloat16),
         lo_v.astype(jnp.bfloat16)], axis=1)

    out = pl.pallas_call(
        _rvq_body,
        grid=(nq, grid_n),
        in_specs=[
            pl.BlockSpec((n, d), lambda q, i: (0, 0)),
            pl.BlockSpec((1, k, d), lambda q, i: (q, 0, 0)),
            pl.BlockSpec((1, 3 * k, d), lambda q, i: (q, 0, 0)),
            pl.BlockSpec((nq, k), lambda q, i: (0, 0)),
        ],
        out_specs=pl.BlockSpec((1, 1, tn), lambda q, i: (q, 0, i)),
        out_shape=jax.ShapeDtypeStruct((nq, 1, n), jnp.int32),
        scratch_shapes=[pltpu.VMEM((n, d), jnp.float32)],
    )(embedding, codebooks, cb_stack, half_csq)

    return jnp.transpose(out.reshape(nq, n))[None, :, :]
